# baseline (speedup 1.0000x reference)
import functools

import jax
import jax.numpy as jnp
from jax import lax
from jax.experimental import pallas as pl
from jax.experimental.pallas import tpu as pltpu
from jax.experimental.pallas import tpu_sc as plsc

N_CHARS = 9
DIM = 1024
MULT_DIM = 8
ROWS = N_CHARS * MULT_DIM
RDIM = DIM * MULT_DIM
HDIM = RDIM // 2

NC = 1
NS = 16
NW = NC * NS
K = 8


def _fuse_body(ce_ref, ne_ref, sep_ref, out_ref):
    sep3 = jnp.broadcast_to(sep_ref[...][:, None, :], (N_CHARS, MULT_DIM, DIM))
    out_ref[0:N_CHARS] = ne_ref[...].reshape(N_CHARS, MULT_DIM, DIM) + sep3
    out_ref[N_CHARS:2 * N_CHARS] = ce_ref[...].reshape(N_CHARS, MULT_DIM, DIM) + sep3


def _build_table(char_emb, not_char_emb, sep_emb):
    return pl.pallas_call(
        _fuse_body,
        out_shape=jax.ShapeDtypeStruct((2 * N_CHARS, MULT_DIM, DIM), jnp.float32),
    )(char_emb, not_char_emb, sep_emb)


def _sc_lookup(table, labels_flat, n_rows):
    n_half = n_rows * 2
    hr_per_w = n_half // NW
    nch = hr_per_w // K

    @functools.partial(
        pl.kernel,
        out_type=jax.ShapeDtypeStruct((n_half, 32, 128), jnp.float32),
        mesh=plsc.VectorSubcoreMesh(core_axis_name="c", subcore_axis_name="s", num_cores=1),
        scratch_types=[
            pltpu.VMEM((hr_per_w,), jnp.int32),
            pltpu.VMEM((hr_per_w,), jnp.int32),
            pltpu.VMEM((K, 32, 128), jnp.float32),
            pltpu.VMEM((K, 32, 128), jnp.float32),
            pltpu.VMEM_SHARED((4 * N_CHARS, 32, 128), jnp.float32),
            pltpu.SemaphoreType.DMA,
            pltpu.SemaphoreType.DMA,
            pltpu.SemaphoreType.DMA,
            pltpu.SemaphoreType.DMA,
        ],
    )
    def body(tbl_hbm, lbl_hbm, out_hbm, lbl_v, idx_v, buf0, buf1, spm, sg0, sg1, so0, so1):
        cid = lax.axis_index("c")
        sid = lax.axis_index("s")
        wid = sid * NC + cid
        base = wid * hr_per_w

        @pl.when(sid == 0)
        def _stage():
            pltpu.sync_copy(tbl_hbm, spm)

        pltpu.sync_copy(lbl_hbm.at[pl.ds(base, hr_per_w)], lbl_v)
        lane = lax.iota(jnp.int32, 16)
        two = jnp.int32(2)
        for i in range(hr_per_w // 16):
            hr = base + i * 16 + lane
            h = hr % 2
            c = lax.div(hr, two) % N_CHARS
            lbl = lbl_v[pl.ds(i * 16, 16)]
            idx_v[pl.ds(i * 16, 16)] = (lbl * N_CHARS + c) * 2 + h
        plsc.subcore_barrier()

        def chunk_pair(j, carry):
            c0 = j * 2
            c1 = j * 2 + 1
            g0 = pltpu.async_copy(spm.at[idx_v.at[pl.ds(c0 * K, K)]], buf0, sg0)
            g1 = pltpu.async_copy(spm.at[idx_v.at[pl.ds(c1 * K, K)]], buf1, sg1)
            g0.wait()
            o0 = pltpu.async_copy(buf0, out_hbm.at[pl.ds(base + c0 * K, K)], so0)
            g1.wait()
            o1 = pltpu.async_copy(buf1, out_hbm.at[pl.ds(base + c1 * K, K)], so1)
            o0.wait()
            o1.wait()
            return carry

        lax.fori_loop(0, nch // 2, chunk_pair, 0)

    return body(table, labels_flat)


def kernel(character_labels, char_emb, not_char_emb, sep_emb):
    b = character_labels.shape[0]
    table = _build_table(char_emb, not_char_emb, sep_emb).reshape(4 * N_CHARS, 32, 128)
    labels_half = jnp.repeat(character_labels.astype(jnp.int32).reshape(b * N_CHARS), 2)
    out = _sc_lookup(table, labels_half, b * N_CHARS)
    return out.reshape(b, ROWS, DIM)


# speedup vs baseline: 1.5161x; 1.5161x over previous
import functools

import jax
import jax.numpy as jnp
from jax import lax
from jax.experimental import pallas as pl
from jax.experimental.pallas import tpu as pltpu
from jax.experimental.pallas import tpu_sc as plsc

N_CHARS = 9
DIM = 1024
MULT_DIM = 8
ROWS = N_CHARS * MULT_DIM
RDIM = DIM * MULT_DIM
HDIM = RDIM // 2

NC = 2
NS = 16
NW = NC * NS
K = 8


def _fuse_body(ce_ref, ne_ref, sep_ref, out_ref):
    sep3 = jnp.broadcast_to(sep_ref[...][:, None, :], (N_CHARS, MULT_DIM, DIM))
    out_ref[0:N_CHARS] = ne_ref[...].reshape(N_CHARS, MULT_DIM, DIM) + sep3
    out_ref[N_CHARS:2 * N_CHARS] = ce_ref[...].reshape(N_CHARS, MULT_DIM, DIM) + sep3


def _build_table(char_emb, not_char_emb, sep_emb):
    return pl.pallas_call(
        _fuse_body,
        out_shape=jax.ShapeDtypeStruct((2 * N_CHARS, MULT_DIM, DIM), jnp.float32),
    )(char_emb, not_char_emb, sep_emb)


def _sc_lookup(table, labels_flat, n_rows):
    n_half = n_rows * 2
    hr_per_w = n_half // NW
    nch = hr_per_w // K

    @functools.partial(
        pl.kernel,
        out_type=jax.ShapeDtypeStruct((n_half, 32, 128), jnp.float32),
        mesh=plsc.VectorSubcoreMesh(core_axis_name="c", subcore_axis_name="s"),
        scratch_types=[
            pltpu.VMEM((hr_per_w,), jnp.int32),
            pltpu.VMEM((hr_per_w,), jnp.int32),
            pltpu.VMEM((K, 32, 128), jnp.float32),
            pltpu.VMEM((K, 32, 128), jnp.float32),
            pltpu.VMEM((K, 32, 128), jnp.float32),
            pltpu.VMEM_SHARED((4 * N_CHARS, 32, 128), jnp.float32),
            pltpu.SemaphoreType.DMA,
            pltpu.SemaphoreType.DMA,
            pltpu.SemaphoreType.DMA,
            pltpu.SemaphoreType.DMA,
            pltpu.SemaphoreType.DMA,
            pltpu.SemaphoreType.DMA,
        ],
    )
    def body(tbl_hbm, lbl_hbm, out_hbm, lbl_v, idx_v, buf0, buf1, buf2, spm,
             sg0, sg1, sg2, so0, so1, so2):
        cid = lax.axis_index("c")
        sid = lax.axis_index("s")
        wid = sid * NC + cid
        base = wid * hr_per_w

        @pl.when(sid == 0)
        def _stage():
            pltpu.sync_copy(tbl_hbm, spm)

        pltpu.sync_copy(lbl_hbm.at[pl.ds(base, hr_per_w)], lbl_v)
        lane = lax.iota(jnp.int32, 16)
        two = jnp.int32(2)
        for i in range(hr_per_w // 16):
            hr = base + i * 16 + lane
            h = hr % 2
            c = lax.div(hr, two) % N_CHARS
            lbl = lbl_v[pl.ds(i * 16, 16)]
            idx_v[pl.ds(i * 16, 16)] = (lbl * N_CHARS + c) * 2 + h
        plsc.subcore_barrier()

        bufs = (buf0, buf1, buf2)
        sgs = (sg0, sg1, sg2)
        sos = (so0, so1, so2)

        def gather(c, slot):
            pltpu.async_copy(spm.at[idx_v.at[pl.ds(c * K, K)]], bufs[slot], sgs[slot])

        def wait_g(slot):
            pltpu.make_async_copy(
                spm.at[idx_v.at[pl.ds(0, K)]], bufs[slot], sgs[slot]).wait()

        def write(c, slot):
            pltpu.async_copy(bufs[slot], out_hbm.at[pl.ds(base + c * K, K)], sos[slot])

        def wait_o(slot):
            pltpu.make_async_copy(
                bufs[slot], out_hbm.at[pl.ds(base, K)], sos[slot]).wait()

        ngrp = nch // 3

        def group(c0, wait_prev_o2, prefetch):
            if wait_prev_o2:
                wait_o(2)
            gather(c0 + 2, 2)
            wait_g(0)
            write(c0, 0)
            wait_o(0)
            if prefetch:
                gather(c0 + 3, 0)
            wait_g(1)
            write(c0 + 1, 1)
            wait_o(1)
            if prefetch:
                gather(c0 + 4, 1)
            wait_g(2)
            write(c0 + 2, 2)

        gather(0, 0)
        gather(1, 1)
        group(0, False, True)

        def mid(g, carry):
            group(g * 3, True, True)
            return carry

        lax.fori_loop(1, ngrp - 1, mid, 0)
        group((ngrp - 1) * 3, True, False)
        wait_o(2)

    return body(table, labels_flat)


def kernel(character_labels, char_emb, not_char_emb, sep_emb):
    b = character_labels.shape[0]
    table = _build_table(char_emb, not_char_emb, sep_emb).reshape(4 * N_CHARS, 32, 128)
    labels_half = jnp.repeat(character_labels.astype(jnp.int32).reshape(b * N_CHARS), 2)
    out = _sc_lookup(table, labels_half, b * N_CHARS)
    return out.reshape(b, ROWS, DIM)


# speedup vs baseline: 1.7582x; 1.1597x over previous
import functools

import jax
import jax.numpy as jnp
from jax import lax
from jax.experimental import pallas as pl
from jax.experimental.pallas import tpu as pltpu
from jax.experimental.pallas import tpu_sc as plsc

N_CHARS = 9
DIM = 1024
MULT_DIM = 8
ROWS = N_CHARS * MULT_DIM
RDIM = DIM * MULT_DIM
HDIM = RDIM // 2

NC = 2
NS = 16
NW = NC * NS
K = 8


def _fuse_body(ce_ref, ne_ref, sep_ref, out_ref):
    sep3 = jnp.broadcast_to(sep_ref[...][:, None, :], (N_CHARS, MULT_DIM, DIM))
    out_ref[0:N_CHARS] = ne_ref[...].reshape(N_CHARS, MULT_DIM, DIM) + sep3
    out_ref[N_CHARS:2 * N_CHARS] = ce_ref[...].reshape(N_CHARS, MULT_DIM, DIM) + sep3


def _build_table(char_emb, not_char_emb, sep_emb):
    return pl.pallas_call(
        _fuse_body,
        out_shape=jax.ShapeDtypeStruct((2 * N_CHARS, MULT_DIM, DIM), jnp.float32),
    )(char_emb, not_char_emb, sep_emb)


def _sc_lookup(table, labels_flat, n_rows):
    n_half = n_rows * 2
    hr_per_w = n_half // NW
    nch = hr_per_w // K

    @functools.partial(
        pl.kernel,
        out_type=jax.ShapeDtypeStruct((n_half, 32, 128), jnp.float32),
        mesh=plsc.VectorSubcoreMesh(core_axis_name="c", subcore_axis_name="s"),
        scratch_types=[
            pltpu.VMEM((hr_per_w,), jnp.int32),
            pltpu.VMEM((hr_per_w,), jnp.int32),
            pltpu.VMEM((K, 32, 128), jnp.float32),
            pltpu.VMEM((K, 32, 128), jnp.float32),
            pltpu.VMEM((K, 32, 128), jnp.float32),
            pltpu.VMEM_SHARED((4 * N_CHARS, 32, 128), jnp.float32),
            pltpu.SemaphoreType.DMA,
            pltpu.SemaphoreType.DMA,
            pltpu.SemaphoreType.DMA,
            pltpu.SemaphoreType.DMA,
            pltpu.SemaphoreType.DMA,
            pltpu.SemaphoreType.DMA,
        ],
    )
    def body(tbl_hbm, lbl_hbm, out_hbm, lbl_v, idx_v, buf0, buf1, buf2, spm,
             sg0, sg1, sg2, so0, so1, so2):
        cid = lax.axis_index("c")
        sid = lax.axis_index("s")
        wid = sid * NC + cid
        base = wid * hr_per_w

        @pl.when(sid == 0)
        def _stage():
            pltpu.sync_copy(tbl_hbm, spm)

        pltpu.sync_copy(lbl_hbm.at[pl.ds(base, hr_per_w)], lbl_v)
        lane = lax.iota(jnp.int32, 16)
        two = jnp.int32(2)
        for i in range(hr_per_w // 16):
            hr = base + i * 16 + lane
            h = hr % 2
            c = lax.div(hr, two) % N_CHARS
            lbl = lbl_v[pl.ds(i * 16, 16)]
            idx_v[pl.ds(i * 16, 16)] = (lbl * N_CHARS + c) * 2 + h
        plsc.subcore_barrier()

        bufs = (buf0, buf1, buf2)
        sgs = (sg0, sg1, sg2)
        sos = (so0, so1, so2)

        def gather(c, slot):
            pltpu.async_copy(spm.at[idx_v.at[pl.ds(c * K, K)]], bufs[slot], sgs[slot])

        def wait_g(slot):
            pltpu.make_async_copy(
                spm.at[idx_v.at[pl.ds(0, K)]], bufs[slot], sgs[slot]).wait()

        def write(c, slot):
            pltpu.async_copy(bufs[slot], out_hbm.at[pl.ds(base + c * K, K)], sos[slot])

        def wait_o(slot):
            pltpu.make_async_copy(
                bufs[slot], out_hbm.at[pl.ds(base, K)], sos[slot]).wait()

        ngrp = nch // 3

        def group(c0, wait_prev_o2, prefetch):
            if wait_prev_o2:
                wait_o(2)
            gather(c0 + 2, 2)
            wait_g(0)
            write(c0, 0)
            wait_o(0)
            if prefetch:
                gather(c0 + 3, 0)
            wait_g(1)
            write(c0 + 1, 1)
            wait_o(1)
            if prefetch:
                gather(c0 + 4, 1)
            wait_g(2)
            write(c0 + 2, 2)

        gather(0, 0)
        gather(1, 1)
        group(0, False, True)

        def mid(g, carry):
            group(g * 3, True, True)
            return carry

        lax.fori_loop(1, ngrp - 1, mid, 0)
        group((ngrp - 1) * 3, True, False)
        wait_o(2)

    return body(table, labels_flat)


BB = 8


def _tc_body(bits_ref, tbl_ref, out_ref):
    i = pl.program_id(0)
    for r in range(BB):
        bits = bits_ref[i * BB + r]
        for c in range(N_CHARS):
            sel = ((bits >> c) & 1) == 1
            lo = c * MULT_DIM
            out_ref[r, lo:lo + MULT_DIM, :] = jnp.where(
                sel, tbl_ref[N_CHARS + c], tbl_ref[c])


def _tc_lookup(bits, table3, b_tc):
    return pl.pallas_call(
        _tc_body,
        grid=(b_tc // BB,),
        in_specs=[
            pl.BlockSpec(memory_space=pltpu.SMEM),
            pl.BlockSpec((2 * N_CHARS, MULT_DIM, DIM), lambda i: (0, 0, 0)),
        ],
        out_specs=pl.BlockSpec((BB, ROWS, DIM), lambda i: (i, 0, 0)),
        out_shape=jax.ShapeDtypeStruct((b_tc, ROWS, DIM), jnp.float32),
    )(bits, table3)


def kernel(character_labels, char_emb, not_char_emb, sep_emb):
    b = character_labels.shape[0]
    b_sc = 256
    b_tc = b - b_sc
    labels = character_labels.astype(jnp.int32)
    table3 = _build_table(char_emb, not_char_emb, sep_emb)
    table_sc = table3.reshape(4 * N_CHARS, 32, 128)
    labels_half = jnp.repeat(labels[b_tc:].reshape(b_sc * N_CHARS), 2)
    out_sc = _sc_lookup(table_sc, labels_half, b_sc * N_CHARS)
    bits = jnp.sum(
        labels[:b_tc] << jnp.arange(N_CHARS, dtype=jnp.int32)[None, :],
        axis=1, dtype=jnp.int32)
    out_tc = _tc_lookup(bits, table3, b_tc)
    return jnp.concatenate([out_tc, out_sc.reshape(b_sc, ROWS, DIM)], axis=0)
